# speedup vs baseline: 3.7495x; 3.7495x over previous
import functools
import jax
import jax.numpy as jnp
from jax import lax
from jax.experimental import pallas as pl
from jax.experimental.pallas import tpu as pltpu
from jax.experimental.pallas import tpu_sc as plsc

_NUM_SC_CORES = 2
_NUM_SC_SUBCORES = 16
_NUM_SC_WORKERS = _NUM_SC_CORES * _NUM_SC_SUBCORES


def _topk_body(K, BK, x_ref, mem_ref, score_ref, idx_ref, qn_ref):
    k = pl.program_id(0)

    @pl.when(k == 0)
    def _():
        x = x_ref[...]
        n = jnp.sqrt(jnp.sum(x * x, axis=1, keepdims=True))
        qn_ref[...] = x / jnp.maximum(n, 1e-12)

    m = mem_ref[...]
    mnorm = jnp.sqrt(jnp.sum(m * m, axis=1, keepdims=True))
    mn = m / jnp.maximum(mnorm, 1e-12)
    scores = lax.dot_general(
        qn_ref[...], mn, (((1,), (1,)), ((), ())),
        preferred_element_type=jnp.float32,
    )
    bmax = jnp.max(scores, axis=1, keepdims=True)
    col = lax.broadcasted_iota(jnp.int32, scores.shape, 1) + k * BK
    bidx = jnp.min(
        jnp.where(scores == bmax, col, K), axis=1, keepdims=True
    )

    @pl.when(k == 0)
    def _():
        score_ref[...] = bmax
        idx_ref[...] = bidx

    @pl.when(k > 0)
    def _():
        prev = score_ref[...]
        better = bmax > prev
        score_ref[...] = jnp.where(better, bmax, prev)
        idx_ref[...] = jnp.where(better, bidx, idx_ref[...])


def _build_topk(Q, D, K, BK, interpret=False):
    assert K % BK == 0
    return pl.pallas_call(
        functools.partial(_topk_body, K, BK),
        grid=(K // BK,),
        in_specs=[
            pl.BlockSpec((Q, D), lambda k: (0, 0)),
            pl.BlockSpec((BK, D), lambda k: (k, 0)),
        ],
        out_specs=[
            pl.BlockSpec((Q, 1), lambda k: (0, 0)),
            pl.BlockSpec((Q, 1), lambda k: (0, 0)),
        ],
        out_shape=[
            jax.ShapeDtypeStruct((Q, 1), jnp.float32),
            jax.ShapeDtypeStruct((Q, 1), jnp.int32),
        ],
        scratch_shapes=[pltpu.VMEM((Q, D), jnp.float32)],
        interpret=interpret,
    )


def _build_sc_gather(Q, D):
    assert Q % (8 * _NUM_SC_WORKERS) == 0
    bpw = Q // _NUM_SC_WORKERS
    mesh = plsc.VectorSubcoreMesh(core_axis_name="c", subcore_axis_name="s")

    @functools.partial(
        pl.kernel,
        mesh=mesh,
        out_type=jax.ShapeDtypeStruct((Q, D), jnp.float32),
        scratch_types=[
            pltpu.VMEM((bpw,), jnp.int32),
            pltpu.VMEM((bpw, D), jnp.float32),
            pltpu.SemaphoreType.DMA,
        ],
    )
    def gather_kernel(table_hbm, idx_hbm, out_hbm, idx_v, rows_v, sem):
        wid = lax.axis_index("s") * _NUM_SC_CORES + lax.axis_index("c")
        base = wid * bpw
        pltpu.sync_copy(idx_hbm.at[pl.ds(base, bpw)], idx_v)
        pltpu.async_copy(table_hbm.at[idx_v], rows_v, sem).wait()
        pltpu.sync_copy(rows_v, out_hbm.at[pl.ds(base, bpw)])

    return gather_kernel


def kernel(x, memory_states):
    Q, D = x.shape
    K = memory_states.shape[0]
    BK = 2000
    best_score, best_idx = _build_topk(Q, D, K, BK)(x, memory_states)
    gather = _build_sc_gather(Q, D)
    retrieved = gather(memory_states, best_idx.reshape(Q))
    return retrieved.reshape(Q, 1, D), best_score
